# Initial kernel scaffold; baseline (speedup 1.0000x reference)
import jax
import jax.numpy as jnp
from jax.experimental import pallas as pl


def kernel(x, edge_index, lin_W0, lin_b0, agg_W0, agg_b0, lin_W1, lin_b1, agg_W1, agg_b1, post_W1, post_b1, post_W2, post_b2):
    raise NotImplementedError("write your pallas kernel here")


# speedup vs baseline: 3.0231x; 3.0231x over previous
import functools

import jax
import jax.numpy as jnp
from jax import lax
from jax.experimental import pallas as pl
from jax.experimental.pallas import tpu as pltpu
from jax.experimental.pallas import tpu_sc as plsc

N = 10000
E = 320000
NR = 4
D = 128
OUT = 64
ROW_BLK = 1000
GRID = N // ROW_BLK

CH = 128
NCH = 2560
E_PAD = NCH * CH
NWORK = 32
CPW = NCH // NWORK
RPS = 632
N_ACC = 16 * RPS


def _weighted_msg(f, c0, c1, c2, w_ref, b_ref):
    p0 = jnp.dot(f, w_ref[0], preferred_element_type=jnp.float32) + b_ref[0:1, :]
    p1 = jnp.dot(f, w_ref[1], preferred_element_type=jnp.float32) + b_ref[1:2, :]
    p2 = jnp.dot(f, w_ref[2], preferred_element_type=jnp.float32) + b_ref[2:3, :]
    return jnp.maximum(c0 * p0 + c1 * p1 + c2 * p2, 0.0)


def _coefs(role_ref):
    r = role_ref[...]
    return 2.0 * r[:, 0:1], r[:, 1:2], r[:, 2:3]


def _update(f, aggr, c0, c1, c2, aw_ref, ab_ref):
    def q(i):
        return (jnp.dot(f, aw_ref[i, :D, :], preferred_element_type=jnp.float32)
                + jnp.dot(aggr, aw_ref[i, D:, :], preferred_element_type=jnp.float32)
                + ab_ref[i:i + 1, :])
    u = jnp.maximum(c0 * q(0) + c1 * q(1) + c2 * q(2), 0.0)
    norm = jnp.sqrt(jnp.sum(u * u, axis=1, keepdims=True))
    return u / jnp.maximum(norm, 1e-12)


def _aggr_from_partials(part_ref, pcnt_ref):
    sums = part_ref[0] + part_ref[1]
    cnt = pcnt_ref[0][:, 0:1] + pcnt_ref[1][:, 0:1]
    return sums / jnp.maximum(cnt, 1.0)


def _msg_kernel(feat_ref, role_ref, w_ref, b_ref, out_ref):
    c0, c1, c2 = _coefs(role_ref)
    out_ref[...] = _weighted_msg(feat_ref[...], c0, c1, c2, w_ref, b_ref)


def _upd_msg_kernel(feat_ref, role_ref, part_ref, pcnt_ref, aw_ref, ab_ref,
                    lw_ref, lb_ref, u_ref, m_ref):
    c0, c1, c2 = _coefs(role_ref)
    aggr = _aggr_from_partials(part_ref, pcnt_ref)
    u = _update(feat_ref[...], aggr, c0, c1, c2, aw_ref, ab_ref)
    u_ref[...] = u
    m_ref[...] = _weighted_msg(u, c0, c1, c2, lw_ref, lb_ref)


def _upd_post_kernel(feat_ref, role_ref, part_ref, pcnt_ref, aw_ref, ab_ref,
                     pw1_ref, pb1_ref, pw2_ref, pb2_ref, out_ref):
    c0, c1, c2 = _coefs(role_ref)
    aggr = _aggr_from_partials(part_ref, pcnt_ref)
    u = _update(feat_ref[...], aggr, c0, c1, c2, aw_ref, ab_ref)
    z = jnp.dot(u, pw1_ref[...], preferred_element_type=jnp.float32) + pb1_ref[...]
    z = jnp.dot(z, pw2_ref[...], preferred_element_type=jnp.float32) + pb2_ref[...]
    zmax = jnp.max(z, axis=1, keepdims=True)
    lse = zmax + jnp.log(jnp.sum(jnp.exp(z - zmax), axis=1, keepdims=True))
    out_ref[...] = z - lse


def _rows(i):
    return (i, 0)


def _full2(i):
    return (0, 0)


def _full3(i):
    return (0, 0, 0)


_msg_call = pl.pallas_call(
    _msg_kernel,
    grid=(GRID,),
    in_specs=[
        pl.BlockSpec((ROW_BLK, D), _rows),
        pl.BlockSpec((ROW_BLK, NR), _rows),
        pl.BlockSpec((NR, D, D), _full3),
        pl.BlockSpec((NR, D), _full2),
    ],
    out_specs=pl.BlockSpec((ROW_BLK, D), _rows),
    out_shape=jax.ShapeDtypeStruct((N, D), jnp.float32),
)

_upd_msg_call = pl.pallas_call(
    _upd_msg_kernel,
    grid=(GRID,),
    in_specs=[
        pl.BlockSpec((ROW_BLK, D), _rows),
        pl.BlockSpec((ROW_BLK, NR), _rows),
        pl.BlockSpec((2, ROW_BLK, D), lambda i: (0, i, 0)),
        pl.BlockSpec((2, ROW_BLK, D), lambda i: (0, i, 0)),
        pl.BlockSpec((NR, 2 * D, D), _full3),
        pl.BlockSpec((NR, D), _full2),
        pl.BlockSpec((NR, D, D), _full3),
        pl.BlockSpec((NR, D), _full2),
    ],
    out_specs=[
        pl.BlockSpec((ROW_BLK, D), _rows),
        pl.BlockSpec((ROW_BLK, D), _rows),
    ],
    out_shape=[
        jax.ShapeDtypeStruct((N, D), jnp.float32),
        jax.ShapeDtypeStruct((N, D), jnp.float32),
    ],
)

_upd_post_call = pl.pallas_call(
    _upd_post_kernel,
    grid=(GRID,),
    in_specs=[
        pl.BlockSpec((ROW_BLK, D), _rows),
        pl.BlockSpec((ROW_BLK, NR), _rows),
        pl.BlockSpec((2, ROW_BLK, D), lambda i: (0, i, 0)),
        pl.BlockSpec((2, ROW_BLK, D), lambda i: (0, i, 0)),
        pl.BlockSpec((NR, 2 * D, D), _full3),
        pl.BlockSpec((NR, D), _full2),
        pl.BlockSpec((D, D), _full2),
        pl.BlockSpec((1, D), _full2),
        pl.BlockSpec((D, OUT), _full2),
        pl.BlockSpec((1, OUT), _full2),
    ],
    out_specs=pl.BlockSpec((ROW_BLK, OUT), _rows),
    out_shape=jax.ShapeDtypeStruct((N, OUT), jnp.float32),
)


def _sc_scatter_body(m_hbm, src_hbm, dst_hbm, zer_hbm,
                     part_out,
                     src_v, dst_v, rows_v, acc_sh, sem):
    c = lax.axis_index("c")
    s = lax.axis_index("s")
    w = c * 16 + s
    base = w * CPW
    pltpu.sync_copy(src_hbm.at[pl.ds(base, CPW)], src_v)
    pltpu.sync_copy(dst_hbm.at[pl.ds(base, CPW)], dst_v)
    pltpu.sync_copy(zer_hbm, acc_sh.at[pl.ds(s * RPS, RPS)])
    plsc.subcore_barrier()

    def step(j, carry):
        pltpu.async_copy(m_hbm.at[src_v.at[j]], rows_v, sem).wait()
        pltpu.sync_copy(rows_v, acc_sh.at[dst_v.at[j]], add=True)
        return carry

    lax.fori_loop(0, CPW, step, 0)
    plsc.subcore_barrier()
    pltpu.sync_copy(acc_sh.at[pl.ds(s * RPS, RPS)],
                    part_out.at[c, pl.ds(s * RPS, RPS)])


@functools.lru_cache(maxsize=1)
def _sc_kernels():
    mesh = plsc.VectorSubcoreMesh(core_axis_name="c", subcore_axis_name="s")
    scatter = pl.kernel(
        _sc_scatter_body,
        mesh=mesh,
        out_type=jax.ShapeDtypeStruct((2, N_ACC, D), jnp.float32),
        scratch_types=[
            pltpu.VMEM((CPW, CH), jnp.int32),
            pltpu.VMEM((CPW, CH), jnp.int32),
            pltpu.VMEM((CH, D), jnp.float32),
            pltpu.VMEM_SHARED((N_ACC, D), jnp.float32),
            pltpu.SemaphoreType.DMA,
        ],
    )
    return scatter


def kernel(x, edge_index, lin_W0, lin_b0, agg_W0, agg_b0,
           lin_W1, lin_b1, agg_W1, agg_b1,
           post_W1, post_b1, post_W2, post_b2):
    roles = x[:, :NR]
    feats = x[:, NR:]
    pad = E_PAD - E
    src2 = jnp.concatenate(
        [edge_index[0], jnp.zeros((pad,), jnp.int32)]).reshape(NCH, CH)
    dst2 = jnp.concatenate(
        [edge_index[1], jnp.full((pad,), N, jnp.int32)]).reshape(NCH, CH)
    zer = jnp.zeros((RPS, D), jnp.float32)

    sc_scatter = _sc_kernels()
    pcnt = sc_scatter(jnp.ones((N, D), jnp.float32), src2, dst2, zer)
    m0 = _msg_call(feats, roles, lin_W0, lin_b0)
    part0 = sc_scatter(m0, src2, dst2, zer)
    u0, m1 = _upd_msg_call(feats, roles, part0, pcnt, agg_W0, agg_b0,
                           lin_W1, lin_b1)
    part1 = sc_scatter(m1, src2, dst2, zer)
    return _upd_post_call(u0, roles, part1, pcnt, agg_W1, agg_b1,
                          post_W1, post_b1.reshape(1, D),
                          post_W2, post_b2.reshape(1, OUT))


# speedup vs baseline: 3.5929x; 1.1885x over previous
import functools

import jax
import jax.numpy as jnp
from jax import lax
from jax.experimental import pallas as pl
from jax.experimental.pallas import tpu as pltpu
from jax.experimental.pallas import tpu_sc as plsc

N = 10000
E = 320000
NR = 4
D = 128
OUT = 64
ROW_BLK = 1000
GRID = N // ROW_BLK

CH = 128
NCH = 2560
E_PAD = NCH * CH
NWORK = 32
CPW = NCH // NWORK
RPS = 632
N_ACC = 16 * RPS


def _weighted_msg(f, c0, c1, c2, w_ref, b_ref):
    p0 = jnp.dot(f, w_ref[0], preferred_element_type=jnp.float32) + b_ref[0:1, :]
    p1 = jnp.dot(f, w_ref[1], preferred_element_type=jnp.float32) + b_ref[1:2, :]
    p2 = jnp.dot(f, w_ref[2], preferred_element_type=jnp.float32) + b_ref[2:3, :]
    return jnp.maximum(c0 * p0 + c1 * p1 + c2 * p2, 0.0)


def _coefs(role_ref):
    r = role_ref[...]
    return 2.0 * r[:, 0:1], r[:, 1:2], r[:, 2:3]


def _update(f, aggr, c0, c1, c2, aw_ref, ab_ref):
    def q(i):
        return (jnp.dot(f, aw_ref[i, :D, :], preferred_element_type=jnp.float32)
                + jnp.dot(aggr, aw_ref[i, D:, :], preferred_element_type=jnp.float32)
                + ab_ref[i:i + 1, :])
    u = jnp.maximum(c0 * q(0) + c1 * q(1) + c2 * q(2), 0.0)
    norm = jnp.sqrt(jnp.sum(u * u, axis=1, keepdims=True))
    return u / jnp.maximum(norm, 1e-12)


def _aggr_from_partials(part_ref, pcnt_ref):
    sums = part_ref[0] + part_ref[1]
    cnt = pcnt_ref[0][:, 0:1] + pcnt_ref[1][:, 0:1]
    return sums / jnp.maximum(cnt, 1.0)


def _msg_kernel(feat_ref, role_ref, w_ref, b_ref, out_ref):
    c0, c1, c2 = _coefs(role_ref)
    out_ref[...] = _weighted_msg(feat_ref[...], c0, c1, c2, w_ref, b_ref)


def _upd_msg_kernel(feat_ref, role_ref, part_ref, pcnt_ref, aw_ref, ab_ref,
                    lw_ref, lb_ref, u_ref, m_ref):
    c0, c1, c2 = _coefs(role_ref)
    aggr = _aggr_from_partials(part_ref, pcnt_ref)
    u = _update(feat_ref[...], aggr, c0, c1, c2, aw_ref, ab_ref)
    u_ref[...] = u
    m_ref[...] = _weighted_msg(u, c0, c1, c2, lw_ref, lb_ref)


def _upd_post_kernel(feat_ref, role_ref, part_ref, pcnt_ref, aw_ref, ab_ref,
                     pw1_ref, pb1_ref, pw2_ref, pb2_ref, out_ref):
    c0, c1, c2 = _coefs(role_ref)
    aggr = _aggr_from_partials(part_ref, pcnt_ref)
    u = _update(feat_ref[...], aggr, c0, c1, c2, aw_ref, ab_ref)
    z = jnp.dot(u, pw1_ref[...], preferred_element_type=jnp.float32) + pb1_ref[...]
    z = jnp.dot(z, pw2_ref[...], preferred_element_type=jnp.float32) + pb2_ref[...]
    zmax = jnp.max(z, axis=1, keepdims=True)
    lse = zmax + jnp.log(jnp.sum(jnp.exp(z - zmax), axis=1, keepdims=True))
    out_ref[...] = z - lse


def _rows(i):
    return (i, 0)


def _full2(i):
    return (0, 0)


def _full3(i):
    return (0, 0, 0)


_msg_call = pl.pallas_call(
    _msg_kernel,
    grid=(GRID,),
    in_specs=[
        pl.BlockSpec((ROW_BLK, D), _rows),
        pl.BlockSpec((ROW_BLK, NR), _rows),
        pl.BlockSpec((NR, D, D), _full3),
        pl.BlockSpec((NR, D), _full2),
    ],
    out_specs=pl.BlockSpec((ROW_BLK, D), _rows),
    out_shape=jax.ShapeDtypeStruct((N, D), jnp.float32),
)

_upd_msg_call = pl.pallas_call(
    _upd_msg_kernel,
    grid=(GRID,),
    in_specs=[
        pl.BlockSpec((ROW_BLK, D), _rows),
        pl.BlockSpec((ROW_BLK, NR), _rows),
        pl.BlockSpec((2, ROW_BLK, D), lambda i: (0, i, 0)),
        pl.BlockSpec((2, ROW_BLK, D), lambda i: (0, i, 0)),
        pl.BlockSpec((NR, 2 * D, D), _full3),
        pl.BlockSpec((NR, D), _full2),
        pl.BlockSpec((NR, D, D), _full3),
        pl.BlockSpec((NR, D), _full2),
    ],
    out_specs=[
        pl.BlockSpec((ROW_BLK, D), _rows),
        pl.BlockSpec((ROW_BLK, D), _rows),
    ],
    out_shape=[
        jax.ShapeDtypeStruct((N, D), jnp.float32),
        jax.ShapeDtypeStruct((N, D), jnp.float32),
    ],
)

_upd_post_call = pl.pallas_call(
    _upd_post_kernel,
    grid=(GRID,),
    in_specs=[
        pl.BlockSpec((ROW_BLK, D), _rows),
        pl.BlockSpec((ROW_BLK, NR), _rows),
        pl.BlockSpec((2, ROW_BLK, D), lambda i: (0, i, 0)),
        pl.BlockSpec((2, ROW_BLK, D), lambda i: (0, i, 0)),
        pl.BlockSpec((NR, 2 * D, D), _full3),
        pl.BlockSpec((NR, D), _full2),
        pl.BlockSpec((D, D), _full2),
        pl.BlockSpec((1, D), _full2),
        pl.BlockSpec((D, OUT), _full2),
        pl.BlockSpec((1, OUT), _full2),
    ],
    out_specs=pl.BlockSpec((ROW_BLK, OUT), _rows),
    out_shape=jax.ShapeDtypeStruct((N, OUT), jnp.float32),
)


def _zero_rows(zer_hbm, acc_sh, s):
    def zstep(r, carry):
        pltpu.sync_copy(zer_hbm, acc_sh.at[pl.ds(s * RPS + r * 8, 8)])
        return carry

    lax.fori_loop(0, RPS // 8, zstep, 0)


def _sc_counts_body(dst_hbm, zer_hbm, ones_hbm,
                    pcnt_out,
                    dst_v, ones_v, cnt_sh, sem):
    c = lax.axis_index("c")
    s = lax.axis_index("s")
    w = c * 16 + s
    base = w * CPW
    pltpu.sync_copy(dst_hbm.at[pl.ds(base, CPW)], dst_v)
    _zero_rows(zer_hbm, cnt_sh, s)
    pltpu.sync_copy(ones_hbm, ones_v)
    plsc.subcore_barrier()

    K_FIRE = 8

    def round_(g, carry):
        descs = []
        for b in range(K_FIRE):
            descs.append(pltpu.async_copy(
                ones_v, cnt_sh.at[dst_v.at[g * K_FIRE + b]], sem, add=True))
        for d in descs:
            d.wait()
        return carry

    lax.fori_loop(0, CPW // K_FIRE, round_, 0)
    plsc.subcore_barrier()
    pltpu.sync_copy(cnt_sh.at[pl.ds(s * RPS, RPS)],
                    pcnt_out.at[c, pl.ds(s * RPS, RPS)])


NBUF = 2
SB = 8


def _sc_scatter_body(m_hbm, src_hbm, dst_hbm, zer_hbm,
                     part_out,
                     src_v, dst_v, rows_v, acc_sh,
                     gsem0, gsem1, ssem0, ssem1):
    gsems = [gsem0, gsem1]
    ssems = [ssem0, ssem1]
    c = lax.axis_index("c")
    s = lax.axis_index("s")
    w = c * 16 + s
    base = w * CPW
    _zero_rows(zer_hbm, acc_sh, s)
    plsc.subcore_barrier()

    def superblock(sb, carry):
        j0 = base + sb * SB
        pltpu.sync_copy(src_hbm.at[pl.ds(j0, SB)], src_v)
        pltpu.sync_copy(dst_hbm.at[pl.ds(j0, SB)], dst_v)

        def gather_start(k, b):
            pltpu.async_copy(m_hbm.at[src_v.at[k]], rows_v.at[b], gsems[b])

        def gather_wait(k, b):
            pltpu.make_async_copy(m_hbm.at[src_v.at[k]], rows_v.at[b],
                                  gsems[b]).wait()

        def scatter_start(k, b):
            pltpu.async_copy(rows_v.at[b], acc_sh.at[dst_v.at[k]],
                             ssems[b], add=True)

        def scatter_wait(k, b):
            pltpu.make_async_copy(rows_v.at[b], acc_sh.at[dst_v.at[k]],
                                  ssems[b]).wait()

        gather_start(0, 0)
        gather_start(1, 1)
        for k in range(SB):
            b = k % 2
            gather_wait(k, b)
            scatter_start(k, b)
            if k + 2 < SB:
                scatter_wait(k, b)
                gather_start(k + 2, b)
        scatter_wait(SB - 2, 0)
        scatter_wait(SB - 1, 1)
        return carry

    lax.fori_loop(0, CPW // SB, superblock, 0)

    plsc.subcore_barrier()
    pltpu.sync_copy(acc_sh.at[pl.ds(s * RPS, RPS)],
                    part_out.at[c, pl.ds(s * RPS, RPS)])


@functools.lru_cache(maxsize=1)
def _sc_kernels():
    mesh = plsc.VectorSubcoreMesh(core_axis_name="c", subcore_axis_name="s")
    counts = pl.kernel(
        _sc_counts_body,
        mesh=mesh,
        out_type=jax.ShapeDtypeStruct((2, N_ACC, D), jnp.float32),
        scratch_types=[
            pltpu.VMEM((CPW, CH), jnp.int32),
            pltpu.VMEM((CH, D), jnp.float32),
            pltpu.VMEM_SHARED((N_ACC, D), jnp.float32),
            pltpu.SemaphoreType.DMA,
        ],
    )
    scatter = pl.kernel(
        _sc_scatter_body,
        mesh=mesh,
        out_type=jax.ShapeDtypeStruct((2, N_ACC, D), jnp.float32),
        scratch_types=[
            pltpu.VMEM((SB, CH), jnp.int32),
            pltpu.VMEM((SB, CH), jnp.int32),
            pltpu.VMEM((NBUF, CH, D), jnp.float32),
            pltpu.VMEM_SHARED((N_ACC, D), jnp.float32),
        ] + [pltpu.SemaphoreType.DMA] * (2 * NBUF),
    )
    return counts, scatter


def kernel(x, edge_index, lin_W0, lin_b0, agg_W0, agg_b0,
           lin_W1, lin_b1, agg_W1, agg_b1,
           post_W1, post_b1, post_W2, post_b2):
    roles = x[:, :NR]
    feats = x[:, NR:]
    pad = E_PAD - E
    src2 = jnp.concatenate(
        [edge_index[0], jnp.zeros((pad,), jnp.int32)]).reshape(NCH, CH)
    dst2 = jnp.concatenate(
        [edge_index[1], jnp.full((pad,), N, jnp.int32)]).reshape(NCH, CH)
    zer = jnp.zeros((8, D), jnp.float32)

    sc_counts, sc_scatter = _sc_kernels()
    pcnt = sc_counts(dst2, zer, jnp.ones((CH, D), jnp.float32))
    m0 = _msg_call(feats, roles, lin_W0, lin_b0)
    part0 = sc_scatter(m0, src2, dst2, zer)
    u0, m1 = _upd_msg_call(feats, roles, part0, pcnt, agg_W0, agg_b0,
                           lin_W1, lin_b1)
    part1 = sc_scatter(m1, src2, dst2, zer)
    return _upd_post_call(u0, roles, part1, pcnt, agg_W1, agg_b1,
                          post_W1, post_b1.reshape(1, D),
                          post_W2, post_b2.reshape(1, OUT))


# speedup vs baseline: 4.6973x; 1.3074x over previous
import functools

import jax
import jax.numpy as jnp
from jax import lax
from jax.experimental import pallas as pl
from jax.experimental.pallas import tpu as pltpu
from jax.experimental.pallas import tpu_sc as plsc

N = 10000
E = 320000
NR = 4
D = 128
OUT = 64
ROW_BLK = 1000
GRID = N // ROW_BLK

CH = 128
NCH = 2560
E_PAD = NCH * CH
NWORK = 32
CPW = NCH // NWORK
RPS = 632
N_ACC = 16 * RPS


def _weighted_msg(f, c0, c1, c2, w_ref, b_ref):
    p0 = jnp.dot(f, w_ref[0], preferred_element_type=jnp.float32) + b_ref[0:1, :]
    p1 = jnp.dot(f, w_ref[1], preferred_element_type=jnp.float32) + b_ref[1:2, :]
    p2 = jnp.dot(f, w_ref[2], preferred_element_type=jnp.float32) + b_ref[2:3, :]
    return jnp.maximum(c0 * p0 + c1 * p1 + c2 * p2, 0.0)


def _coefs(role_ref):
    r = role_ref[...]
    return 2.0 * r[:, 0:1], r[:, 1:2], r[:, 2:3]


def _update(f, aggr, c0, c1, c2, aw_ref, ab_ref):
    def q(i):
        return (jnp.dot(f, aw_ref[i, :D, :], preferred_element_type=jnp.float32)
                + jnp.dot(aggr, aw_ref[i, D:, :], preferred_element_type=jnp.float32)
                + ab_ref[i:i + 1, :])
    u = jnp.maximum(c0 * q(0) + c1 * q(1) + c2 * q(2), 0.0)
    norm = jnp.sqrt(jnp.sum(u * u, axis=1, keepdims=True))
    return u / jnp.maximum(norm, 1e-12)


def _aggr_from_partials(part_ref, pcnt_ref):
    sums = part_ref[0]
    cnt = pcnt_ref[1][:, 0:1]
    return sums / jnp.maximum(cnt, 1.0)


def _msg_kernel(feat_ref, role_ref, w_ref, b_ref, out_ref):
    c0, c1, c2 = _coefs(role_ref)
    out_ref[...] = _weighted_msg(feat_ref[...], c0, c1, c2, w_ref, b_ref)


def _upd_msg_kernel(feat_ref, role_ref, part_ref, pcnt_ref, aw_ref, ab_ref,
                    lw_ref, lb_ref, u_ref, m_ref):
    c0, c1, c2 = _coefs(role_ref)
    aggr = _aggr_from_partials(part_ref, pcnt_ref)
    u = _update(feat_ref[...], aggr, c0, c1, c2, aw_ref, ab_ref)
    u_ref[...] = u
    m_ref[...] = _weighted_msg(u, c0, c1, c2, lw_ref, lb_ref)


def _upd_post_kernel(feat_ref, role_ref, part_ref, pcnt_ref, aw_ref, ab_ref,
                     pw1_ref, pb1_ref, pw2_ref, pb2_ref, out_ref):
    c0, c1, c2 = _coefs(role_ref)
    aggr = _aggr_from_partials(part_ref, pcnt_ref)
    u = _update(feat_ref[...], aggr, c0, c1, c2, aw_ref, ab_ref)
    z = jnp.dot(u, pw1_ref[...], preferred_element_type=jnp.float32) + pb1_ref[...]
    z = jnp.dot(z, pw2_ref[...], preferred_element_type=jnp.float32) + pb2_ref[...]
    zmax = jnp.max(z, axis=1, keepdims=True)
    lse = zmax + jnp.log(jnp.sum(jnp.exp(z - zmax), axis=1, keepdims=True))
    out_ref[...] = z - lse


def _rows(i):
    return (i, 0)


def _full2(i):
    return (0, 0)


def _full3(i):
    return (0, 0, 0)


_msg_call = pl.pallas_call(
    _msg_kernel,
    grid=(GRID,),
    in_specs=[
        pl.BlockSpec((ROW_BLK, D), _rows),
        pl.BlockSpec((ROW_BLK, NR), _rows),
        pl.BlockSpec((NR, D, D), _full3),
        pl.BlockSpec((NR, D), _full2),
    ],
    out_specs=pl.BlockSpec((ROW_BLK, D), _rows),
    out_shape=jax.ShapeDtypeStruct((N, D), jnp.float32),
)

_upd_msg_call = pl.pallas_call(
    _upd_msg_kernel,
    grid=(GRID,),
    in_specs=[
        pl.BlockSpec((ROW_BLK, D), _rows),
        pl.BlockSpec((ROW_BLK, NR), _rows),
        pl.BlockSpec((2, ROW_BLK, D), lambda i: (0, i, 0)),
        pl.BlockSpec((2, ROW_BLK, D), lambda i: (0, i, 0)),
        pl.BlockSpec((NR, 2 * D, D), _full3),
        pl.BlockSpec((NR, D), _full2),
        pl.BlockSpec((NR, D, D), _full3),
        pl.BlockSpec((NR, D), _full2),
    ],
    out_specs=[
        pl.BlockSpec((ROW_BLK, D), _rows),
        pl.BlockSpec((ROW_BLK, D), _rows),
    ],
    out_shape=[
        jax.ShapeDtypeStruct((N, D), jnp.float32),
        jax.ShapeDtypeStruct((N, D), jnp.float32),
    ],
)

_upd_post_call = pl.pallas_call(
    _upd_post_kernel,
    grid=(GRID,),
    in_specs=[
        pl.BlockSpec((ROW_BLK, D), _rows),
        pl.BlockSpec((ROW_BLK, NR), _rows),
        pl.BlockSpec((1, ROW_BLK, D), lambda i: (0, i, 0)),
        pl.BlockSpec((2, ROW_BLK, D), lambda i: (0, i, 0)),
        pl.BlockSpec((NR, 2 * D, D), _full3),
        pl.BlockSpec((NR, D), _full2),
        pl.BlockSpec((D, D), _full2),
        pl.BlockSpec((1, D), _full2),
        pl.BlockSpec((D, OUT), _full2),
        pl.BlockSpec((1, OUT), _full2),
    ],
    out_specs=pl.BlockSpec((ROW_BLK, OUT), _rows),
    out_shape=jax.ShapeDtypeStruct((N, OUT), jnp.float32),
)


NBUF = 2
CPW1 = NCH // 16
NSB = CPW1 // 8
SB = 8
K_FIRE = 8


def _gather_scatter_loop(m_hbm, src_hbm, dst_hbm, src_v, dst_v, rows_v,
                         acc_sh, gsems, ssems, base):
    def superblock(sb, carry):
        j0 = base + sb * SB
        pltpu.sync_copy(src_hbm.at[pl.ds(j0, SB)], src_v)
        pltpu.sync_copy(dst_hbm.at[pl.ds(j0, SB)], dst_v)

        def gather_start(k, b):
            pltpu.async_copy(m_hbm.at[src_v.at[k]], rows_v.at[b], gsems[b])

        def gather_wait(k, b):
            pltpu.make_async_copy(m_hbm.at[src_v.at[k]], rows_v.at[b],
                                  gsems[b]).wait()

        def scatter_start(k, b):
            pltpu.async_copy(rows_v.at[b], acc_sh.at[dst_v.at[k]],
                             ssems[b], add=True)

        def scatter_wait(k, b):
            pltpu.make_async_copy(rows_v.at[b], acc_sh.at[dst_v.at[k]],
                                  ssems[b]).wait()

        gather_start(0, 0)
        gather_start(1, 1)
        for k in range(SB):
            b = k % 2
            gather_wait(k, b)
            scatter_start(k, b)
            if k + 2 < SB:
                scatter_wait(k, b)
                gather_start(k + 2, b)
        scatter_wait(SB - 2, 0)
        scatter_wait(SB - 1, 1)
        return carry

    lax.fori_loop(0, NSB, superblock, 0)


def _counts_loop(dst_hbm, dst_v, ones_row, acc_sh, sem, base):
    def round_(g, carry):
        j0 = base + g * K_FIRE
        pltpu.sync_copy(dst_hbm.at[pl.ds(j0, K_FIRE)], dst_v)
        descs = []
        for b in range(K_FIRE):
            descs.append(pltpu.async_copy(
                ones_row, acc_sh.at[dst_v.at[b]], sem, add=True))
        for d in descs:
            d.wait()
        return carry

    lax.fori_loop(0, CPW1 // K_FIRE, round_, 0)


def _sc_pass0_body(m_hbm, src_hbm, dst_hbm, zer_hbm, ones_hbm,
                   part_out,
                   src_v, dst_v, rows_v, acc_sh,
                   gsem0, gsem1, ssem0, ssem1):
    c = lax.axis_index("c")
    s = lax.axis_index("s")
    base = s * CPW1
    pltpu.sync_copy(zer_hbm, acc_sh.at[pl.ds(s * RPS, RPS)])

    @pl.when(c == 1)
    def _():
        pltpu.sync_copy(ones_hbm, rows_v.at[0])

    plsc.subcore_barrier()

    @pl.when(c == 0)
    def _():
        _gather_scatter_loop(m_hbm, src_hbm, dst_hbm, src_v, dst_v, rows_v,
                             acc_sh, [gsem0, gsem1], [ssem0, ssem1], base)

    @pl.when(c == 1)
    def _():
        _counts_loop(dst_hbm, dst_v, rows_v.at[0], acc_sh, gsem0, base)

    plsc.subcore_barrier()
    pltpu.sync_copy(acc_sh.at[pl.ds(s * RPS, RPS)],
                    part_out.at[c, pl.ds(s * RPS, RPS)])


def _sc_pass1_body(m_hbm, src_hbm, dst_hbm, zer_hbm,
                   part_out,
                   src_v, dst_v, rows_v, acc_sh,
                   gsem0, gsem1, ssem0, ssem1):
    c = lax.axis_index("c")
    s = lax.axis_index("s")
    base = s * CPW1

    @pl.when(c == 0)
    def _():
        pltpu.sync_copy(zer_hbm, acc_sh.at[pl.ds(s * RPS, RPS)])

    plsc.subcore_barrier()

    @pl.when(c == 0)
    def _():
        _gather_scatter_loop(m_hbm, src_hbm, dst_hbm, src_v, dst_v, rows_v,
                             acc_sh, [gsem0, gsem1], [ssem0, ssem1], base)

    plsc.subcore_barrier()

    @pl.when(c == 0)
    def _():
        pltpu.sync_copy(acc_sh.at[pl.ds(s * RPS, RPS)],
                        part_out.at[0, pl.ds(s * RPS, RPS)])


@functools.lru_cache(maxsize=1)
def _sc_kernels():
    mesh = plsc.VectorSubcoreMesh(core_axis_name="c", subcore_axis_name="s")
    common_scratch = [
        pltpu.VMEM((SB, CH), jnp.int32),
        pltpu.VMEM((SB, CH), jnp.int32),
        pltpu.VMEM((NBUF, CH, D), jnp.float32),
        pltpu.VMEM_SHARED((N_ACC, D), jnp.float32),
    ] + [pltpu.SemaphoreType.DMA] * (2 * NBUF)
    pass0 = pl.kernel(
        _sc_pass0_body,
        mesh=mesh,
        out_type=jax.ShapeDtypeStruct((2, N_ACC, D), jnp.float32),
        scratch_types=list(common_scratch),
    )
    pass1 = pl.kernel(
        _sc_pass1_body,
        mesh=mesh,
        out_type=jax.ShapeDtypeStruct((1, N_ACC, D), jnp.float32),
        scratch_types=list(common_scratch),
    )
    return pass0, pass1


def kernel(x, edge_index, lin_W0, lin_b0, agg_W0, agg_b0,
           lin_W1, lin_b1, agg_W1, agg_b1,
           post_W1, post_b1, post_W2, post_b2):
    roles = x[:, :NR]
    feats = x[:, NR:]
    pad = E_PAD - E
    src2 = jnp.concatenate(
        [edge_index[0], jnp.zeros((pad,), jnp.int32)]).reshape(NCH, CH)
    dst2 = jnp.concatenate(
        [edge_index[1], jnp.full((pad,), N, jnp.int32)]).reshape(NCH, CH)
    zer = jnp.zeros((RPS, D), jnp.float32)
    ones = jnp.ones((CH, D), jnp.float32)

    sc_pass0, sc_pass1 = _sc_kernels()
    m0 = _msg_call(feats, roles, lin_W0, lin_b0)
    part0 = sc_pass0(m0, src2, dst2, zer, ones)
    u0, m1 = _upd_msg_call(feats, roles, part0, part0, agg_W0, agg_b0,
                           lin_W1, lin_b1)
    part1 = sc_pass1(m1, src2, dst2, zer)
    return _upd_post_call(u0, roles, part1, part0, agg_W1, agg_b1,
                          post_W1, post_b1.reshape(1, D),
                          post_W2, post_b2.reshape(1, OUT))


# speedup vs baseline: 4.6980x; 1.0001x over previous
import functools

import jax
import jax.numpy as jnp
from jax import lax
from jax.experimental import pallas as pl
from jax.experimental.pallas import tpu as pltpu
from jax.experimental.pallas import tpu_sc as plsc

N = 10000
E = 320000
NR = 4
D = 128
OUT = 64
ROW_BLK = 1000
GRID = N // ROW_BLK

CH = 128
NCH = 2560
E_PAD = NCH * CH
NWORK = 32
CPW = NCH // NWORK
RPS = 632
N_ACC = 16 * RPS


def _weighted_msg(f, c0, c1, c2, w_ref, b_ref):
    p0 = jnp.dot(f, w_ref[0], preferred_element_type=jnp.float32) + b_ref[0:1, :]
    p1 = jnp.dot(f, w_ref[1], preferred_element_type=jnp.float32) + b_ref[1:2, :]
    p2 = jnp.dot(f, w_ref[2], preferred_element_type=jnp.float32) + b_ref[2:3, :]
    return jnp.maximum(c0 * p0 + c1 * p1 + c2 * p2, 0.0)


def _coefs(role_ref):
    r = role_ref[...]
    return 2.0 * r[:, 0:1], r[:, 1:2], r[:, 2:3]


def _update(f, aggr, c0, c1, c2, aw_ref, ab_ref):
    def q(i):
        return (jnp.dot(f, aw_ref[i, :D, :], preferred_element_type=jnp.float32)
                + jnp.dot(aggr, aw_ref[i, D:, :], preferred_element_type=jnp.float32)
                + ab_ref[i:i + 1, :])
    u = jnp.maximum(c0 * q(0) + c1 * q(1) + c2 * q(2), 0.0)
    norm = jnp.sqrt(jnp.sum(u * u, axis=1, keepdims=True))
    return u / jnp.maximum(norm, 1e-12)


def _aggr_from_partials(part_ref, pcnt_ref):
    sums = part_ref[0]
    cnt = pcnt_ref[1][:, 0:1]
    return sums / jnp.maximum(cnt, 1.0)


def _msg_kernel(feat_ref, role_ref, w_ref, b_ref, out_ref):
    c0, c1, c2 = _coefs(role_ref)
    out_ref[...] = _weighted_msg(feat_ref[...], c0, c1, c2, w_ref, b_ref)


def _upd_msg_kernel(feat_ref, role_ref, part_ref, pcnt_ref, aw_ref, ab_ref,
                    lw_ref, lb_ref, u_ref, m_ref):
    c0, c1, c2 = _coefs(role_ref)
    aggr = _aggr_from_partials(part_ref, pcnt_ref)
    u = _update(feat_ref[...], aggr, c0, c1, c2, aw_ref, ab_ref)
    u_ref[...] = u
    m_ref[...] = _weighted_msg(u, c0, c1, c2, lw_ref, lb_ref)


def _upd_post_kernel(feat_ref, role_ref, part_ref, pcnt_ref, aw_ref, ab_ref,
                     pw1_ref, pb1_ref, pw2_ref, pb2_ref, out_ref):
    c0, c1, c2 = _coefs(role_ref)
    aggr = _aggr_from_partials(part_ref, pcnt_ref)
    u = _update(feat_ref[...], aggr, c0, c1, c2, aw_ref, ab_ref)
    z = jnp.dot(u, pw1_ref[...], preferred_element_type=jnp.float32) + pb1_ref[...]
    z = jnp.dot(z, pw2_ref[...], preferred_element_type=jnp.float32) + pb2_ref[...]
    zmax = jnp.max(z, axis=1, keepdims=True)
    lse = zmax + jnp.log(jnp.sum(jnp.exp(z - zmax), axis=1, keepdims=True))
    out_ref[...] = z - lse


def _rows(i):
    return (i, 0)


def _full2(i):
    return (0, 0)


def _full3(i):
    return (0, 0, 0)


_msg_call = pl.pallas_call(
    _msg_kernel,
    grid=(GRID,),
    in_specs=[
        pl.BlockSpec((ROW_BLK, D), _rows),
        pl.BlockSpec((ROW_BLK, NR), _rows),
        pl.BlockSpec((NR, D, D), _full3),
        pl.BlockSpec((NR, D), _full2),
    ],
    out_specs=pl.BlockSpec((ROW_BLK, D), _rows),
    out_shape=jax.ShapeDtypeStruct((N, D), jnp.float32),
)

_upd_msg_call = pl.pallas_call(
    _upd_msg_kernel,
    grid=(GRID,),
    in_specs=[
        pl.BlockSpec((ROW_BLK, D), _rows),
        pl.BlockSpec((ROW_BLK, NR), _rows),
        pl.BlockSpec((2, ROW_BLK, D), lambda i: (0, i, 0)),
        pl.BlockSpec((2, ROW_BLK, D), lambda i: (0, i, 0)),
        pl.BlockSpec((NR, 2 * D, D), _full3),
        pl.BlockSpec((NR, D), _full2),
        pl.BlockSpec((NR, D, D), _full3),
        pl.BlockSpec((NR, D), _full2),
    ],
    out_specs=[
        pl.BlockSpec((ROW_BLK, D), _rows),
        pl.BlockSpec((ROW_BLK, D), _rows),
    ],
    out_shape=[
        jax.ShapeDtypeStruct((N, D), jnp.float32),
        jax.ShapeDtypeStruct((N, D), jnp.float32),
    ],
)

_upd_post_call = pl.pallas_call(
    _upd_post_kernel,
    grid=(GRID,),
    in_specs=[
        pl.BlockSpec((ROW_BLK, D), _rows),
        pl.BlockSpec((ROW_BLK, NR), _rows),
        pl.BlockSpec((1, ROW_BLK, D), lambda i: (0, i, 0)),
        pl.BlockSpec((2, ROW_BLK, D), lambda i: (0, i, 0)),
        pl.BlockSpec((NR, 2 * D, D), _full3),
        pl.BlockSpec((NR, D), _full2),
        pl.BlockSpec((D, D), _full2),
        pl.BlockSpec((1, D), _full2),
        pl.BlockSpec((D, OUT), _full2),
        pl.BlockSpec((1, OUT), _full2),
    ],
    out_specs=pl.BlockSpec((ROW_BLK, OUT), _rows),
    out_shape=jax.ShapeDtypeStruct((N, OUT), jnp.float32),
)


NBUF = 2
CPW1 = NCH // 16
NSB = CPW1 // 8
SB = 8
K_FIRE = 8
QS = 4
QL = CH // QS


def _gather_scatter_loop_n(m_hbm, src_hbm, dst_hbm, src_v, dst_v, rows_v,
                           acc_sh, gsems, ssems, base, n_sb):
    def superblock(sb, carry):
        j0 = base + sb * SB
        pltpu.sync_copy(src_hbm.at[pl.ds(j0, SB)], src_v)
        pltpu.sync_copy(dst_hbm.at[pl.ds(j0, SB)], dst_v)

        def gather_start(k, b):
            for q in range(QS):
                pltpu.async_copy(m_hbm.at[src_v.at[k, pl.ds(q * QL, QL)]],
                                 rows_v.at[b, pl.ds(q * QL, QL)], gsems[b])

        def gather_wait(k, b):
            for q in range(QS):
                pltpu.make_async_copy(m_hbm.at[src_v.at[k, pl.ds(q * QL, QL)]],
                                      rows_v.at[b, pl.ds(q * QL, QL)],
                                      gsems[b]).wait()

        def scatter_start(k, b):
            pltpu.async_copy(rows_v.at[b], acc_sh.at[dst_v.at[k]],
                             ssems[b], add=True)

        def scatter_wait(k, b):
            pltpu.make_async_copy(rows_v.at[b], acc_sh.at[dst_v.at[k]],
                                  ssems[b]).wait()

        gather_start(0, 0)
        gather_start(1, 1)
        for k in range(SB):
            b = k % 2
            gather_wait(k, b)
            scatter_start(k, b)
            if k + 2 < SB:
                scatter_wait(k, b)
                gather_start(k + 2, b)
        scatter_wait(SB - 2, 0)
        scatter_wait(SB - 1, 1)
        return carry

    lax.fori_loop(0, n_sb, superblock, 0)


def _counts_loop(dst_hbm, dst_v, ones_row, acc_sh, sem, base):
    def round_(g, carry):
        j0 = base + g * K_FIRE
        pltpu.sync_copy(dst_hbm.at[pl.ds(j0, K_FIRE)], dst_v)
        descs = []
        for b in range(K_FIRE):
            descs.append(pltpu.async_copy(
                ones_row, acc_sh.at[dst_v.at[b]], sem, add=True))
        for d in descs:
            d.wait()
        return carry

    lax.fori_loop(0, CPW1 // K_FIRE, round_, 0)


def _sc_pass0_body(m_hbm, src_hbm, dst_hbm, zer_hbm, ones_hbm,
                   part_out,
                   src_v, dst_v, rows_v, acc_sh,
                   gsem0, gsem1, ssem0, ssem1):
    c = lax.axis_index("c")
    s = lax.axis_index("s")
    base = s * CPW1
    pltpu.sync_copy(zer_hbm, acc_sh.at[pl.ds(s * RPS, RPS)])

    @pl.when(c == 1)
    def _():
        pltpu.sync_copy(ones_hbm, rows_v.at[0])

    plsc.subcore_barrier()

    @pl.when(c == 0)
    def _():
        _gather_scatter_loop_n(m_hbm, src_hbm, dst_hbm, src_v, dst_v, rows_v,
                               acc_sh, [gsem0, gsem1], [ssem0, ssem1],
                               base, NSB)

    @pl.when(c == 1)
    def _():
        _counts_loop(dst_hbm, dst_v, rows_v.at[0], acc_sh, gsem0, base)

    plsc.subcore_barrier()
    pltpu.sync_copy(acc_sh.at[pl.ds(s * RPS, RPS)],
                    part_out.at[c, pl.ds(s * RPS, RPS)])


def _sc_pass1_body(m_hbm, src_hbm, dst_hbm, zer_hbm,
                   part_out,
                   src_v, dst_v, rows_v, acc_sh,
                   gsem0, gsem1, ssem0, ssem1):
    c = lax.axis_index("c")
    s = lax.axis_index("s")
    base = s * CPW1

    @pl.when(c == 0)
    def _():
        pltpu.sync_copy(zer_hbm, acc_sh.at[pl.ds(s * RPS, RPS)])

    plsc.subcore_barrier()

    @pl.when(c == 0)
    def _():
        _gather_scatter_loop_n(m_hbm, src_hbm, dst_hbm, src_v, dst_v, rows_v,
                               acc_sh, [gsem0, gsem1], [ssem0, ssem1],
                               base, NSB)

    plsc.subcore_barrier()

    @pl.when(c == 0)
    def _():
        pltpu.sync_copy(acc_sh.at[pl.ds(s * RPS, RPS)],
                        part_out.at[0, pl.ds(s * RPS, RPS)])


@functools.lru_cache(maxsize=1)
def _sc_kernels():
    mesh = plsc.VectorSubcoreMesh(core_axis_name="c", subcore_axis_name="s")
    common_scratch = [
        pltpu.VMEM((SB, CH), jnp.int32),
        pltpu.VMEM((SB, CH), jnp.int32),
        pltpu.VMEM((NBUF, CH, D), jnp.float32),
        pltpu.VMEM_SHARED((N_ACC, D), jnp.float32),
    ] + [pltpu.SemaphoreType.DMA] * (2 * NBUF)
    pass0 = pl.kernel(
        _sc_pass0_body,
        mesh=mesh,
        out_type=jax.ShapeDtypeStruct((2, N_ACC, D), jnp.float32),
        scratch_types=list(common_scratch),
    )
    pass1 = pl.kernel(
        _sc_pass1_body,
        mesh=mesh,
        out_type=jax.ShapeDtypeStruct((1, N_ACC, D), jnp.float32),
        scratch_types=list(common_scratch),
    )
    return pass0, pass1


def kernel(x, edge_index, lin_W0, lin_b0, agg_W0, agg_b0,
           lin_W1, lin_b1, agg_W1, agg_b1,
           post_W1, post_b1, post_W2, post_b2):
    roles = x[:, :NR]
    feats = x[:, NR:]
    pad = E_PAD - E
    src2 = jnp.concatenate(
        [edge_index[0], jnp.zeros((pad,), jnp.int32)]).reshape(NCH, CH)
    dst2 = jnp.concatenate(
        [edge_index[1], jnp.full((pad,), N, jnp.int32)]).reshape(NCH, CH)
    zer = jnp.zeros((RPS, D), jnp.float32)
    ones = jnp.ones((CH, D), jnp.float32)

    sc_pass0, sc_pass1 = _sc_kernels()
    m0 = _msg_call(feats, roles, lin_W0, lin_b0)
    part0 = sc_pass0(m0, src2, dst2, zer, ones)
    u0, m1 = _upd_msg_call(feats, roles, part0, part0, agg_W0, agg_b0,
                           lin_W1, lin_b1)
    part1 = sc_pass1(m1, src2, dst2, zer)
    return _upd_post_call(u0, roles, part1, part0, agg_W1, agg_b1,
                          post_W1, post_b1.reshape(1, D),
                          post_W2, post_b2.reshape(1, OUT))
